# baseline (speedup 1.0000x reference)
import jax
import jax.numpy as jnp
from jax import lax
from jax.experimental import pallas as pl
from jax.experimental.pallas import tpu as pltpu
from jax.experimental.pallas import tpu_sc as plsc

VOCAB = 1000000
EMBED = 64
EPAD = 72
B = 16384
L = 50

NC = 2
NS = 16
NW = NC * NS

BH = B // 128
BH_W = BH // NW
ROWS_W = B // NW
NUNIT = BH_W * L


def _embed_body(weight_hbm, x_hbm, out_hbm, x_v, xt_v, g_v, t_v, semg, sems):
    wid = lax.axis_index("s") * NC + lax.axis_index("c")
    base = wid * ROWS_W
    lanes = lax.iota(jnp.int32, 16)

    pltpu.sync_copy(x_hbm.at[pl.ds(base, ROWS_W)], x_v)

    def idx_t(u, _):
        bb = u // L
        l = u - bb * L
        col = jnp.full((16,), l, jnp.int32)
        for b0 in range(8):
            rows = lanes + (bb * 128 + b0 * 16)
            v = plsc.load_gather(x_v, [rows, col])
            xt_v[bb, l, pl.ds(b0 * 16, 16)] = v
        return 0

    lax.fori_loop(0, NUNIT, idx_t, 0)

    def transpose_unit(gbuf, tbuf):
        buf_c = jnp.full((16,), tbuf, jnp.int32)
        eh_c = [(lanes + e0) // 8 for e0 in (0, 16, 32, 48)]
        el_c = [(lanes + e0) % 8 for e0 in (0, 16, 32, 48)]

        def b_step(b4, _):
            for db in range(8):
                b = b4 * 8 + db
                b_c = jnp.full((16,), b, jnp.int32)
                for k in range(4):
                    v = g_v[gbuf, b, pl.ds(k * 16, 16)]
                    plsc.store_scatter(t_v, [buf_c, eh_c[k], el_c[k], b_c], v)
            return 0

        lax.fori_loop(0, 16, b_step, 0)

    def fire_gather(u, gbuf, sem):
        bb = u // L
        l = u - bb * L
        pltpu.async_copy(
            weight_hbm.at[xt_v.at[bb, l]],
            g_v.at[gbuf],
            sem,
        )

    def drain_gather(gbuf, sem):
        pltpu.make_async_copy(
            weight_hbm.at[pl.ds(0, 128)], g_v.at[gbuf], sem
        ).wait()

    def fire_scatter(u, tbuf, sem):
        bb = u // L
        l = u - bb * L
        pltpu.async_copy(
            t_v.at[tbuf, :, :, pl.ds(0, 128)],
            out_hbm.at[l, :, wid * BH_W + bb],
            sem,
        )

    def wait_scatter(u, tbuf, sem):
        bb = u // L
        l = u - bb * L
        pltpu.make_async_copy(
            t_v.at[tbuf, :, :, pl.ds(0, 128)],
            out_hbm.at[l, :, wid * BH_W + bb],
            sem,
        ).wait()

    fire_gather(0, 0, semg.at[0])
    fire_gather(1, 1, semg.at[1])

    drain_gather(0, semg.at[0])
    transpose_unit(0, 0)
    fire_gather(2, 0, semg.at[0])
    fire_scatter(0, 0, sems.at[0])
    drain_gather(1, semg.at[1])
    transpose_unit(1, 1)
    fire_gather(3, 1, semg.at[1])
    fire_scatter(1, 1, sems.at[1])

    def pair(t, _):
        u = 2 * t + 2
        drain_gather(0, semg.at[0])
        wait_scatter(u - 2, 0, sems.at[0])
        transpose_unit(0, 0)
        fire_gather(u + 2, 0, semg.at[0])
        fire_scatter(u, 0, sems.at[0])
        drain_gather(1, semg.at[1])
        wait_scatter(u - 1, 1, sems.at[1])
        transpose_unit(1, 1)
        fire_gather(u + 3, 1, semg.at[1])
        fire_scatter(u + 1, 1, sems.at[1])
        return 0

    lax.fori_loop(0, (NUNIT - 4) // 2, pair, 0)

    drain_gather(0, semg.at[0])
    wait_scatter(NUNIT - 4, 0, sems.at[0])
    transpose_unit(0, 0)
    fire_scatter(NUNIT - 2, 0, sems.at[0])
    drain_gather(1, semg.at[1])
    wait_scatter(NUNIT - 3, 1, sems.at[1])
    transpose_unit(1, 1)
    fire_scatter(NUNIT - 1, 1, sems.at[1])
    wait_scatter(NUNIT - 2, 0, sems.at[0])
    wait_scatter(NUNIT - 1, 1, sems.at[1])


@jax.jit
def kernel(x, weight):
    wp = jnp.pad(weight, ((0, 0), (0, EPAD - EMBED)))
    mesh = plsc.VectorSubcoreMesh(core_axis_name="c", subcore_axis_name="s")
    out = pl.kernel(
        _embed_body,
        out_type=jax.ShapeDtypeStruct((L, 8, BH, 8, 128), jnp.float32),
        mesh=mesh,
        scratch_types=[
            pltpu.VMEM((ROWS_W, L), jnp.int32),
            pltpu.VMEM((BH_W, L, 128), jnp.int32),
            pltpu.VMEM((2, 128, EPAD), jnp.float32),
            pltpu.VMEM((2, 8, 8, 129), jnp.float32),
            pltpu.SemaphoreType.DMA((2,)),
            pltpu.SemaphoreType.DMA((2,)),
        ],
        compiler_params=pltpu.CompilerParams(
            use_tc_tiling_on_sc=False, needs_layout_passes=False
        ),
    )(wp, x)
    return out.transpose(2, 4, 0, 1, 3).reshape(B, L, EMBED)


# speedup vs baseline: 1.7127x; 1.7127x over previous
import jax
import jax.numpy as jnp
from jax import lax
from jax.experimental import pallas as pl
from jax.experimental.pallas import tpu as pltpu
from jax.experimental.pallas import tpu_sc as plsc

VOCAB = 1000000
EMBED = 64
EPAD = 128
B = 16384
L = 50

NC = 2
NS = 16
NW = NC * NS

BH = B // 128
BH_W = BH // NW
ROWS_W = B // NW
NUNIT = BH_W * L


def _embed_body(weight_hbm, x_hbm, out_hbm, x_v, xt_v, g_v, t_v, semg, sems):
    wid = lax.axis_index("s") * NC + lax.axis_index("c")
    base = wid * ROWS_W
    lanes = lax.iota(jnp.int32, 16)

    pltpu.sync_copy(x_hbm.at[pl.ds(base, ROWS_W)], x_v)

    def idx_t(u, _):
        bb = u // L
        l = u - bb * L
        col = jnp.full((16,), l, jnp.int32)
        for b0 in range(8):
            rows = lanes + (bb * 128 + b0 * 16)
            v = plsc.load_gather(x_v, [rows, col])
            xt_v[bb, l, pl.ds(b0 * 16, 16)] = v
        return 0

    lax.fori_loop(0, NUNIT, idx_t, 0)

    def transpose_unit(gbuf, tbuf):
        buf_c = jnp.full((16,), tbuf, jnp.int32)
        eh_c = [(lanes + e0) // 8 for e0 in (0, 16, 32, 48)]
        el_c = [(lanes + e0) % 8 for e0 in (0, 16, 32, 48)]

        def loads(b):
            return [g_v[gbuf, b, pl.ds(k * 16, 16)] for k in range(4)]

        def stores(b, vs):
            b_c = jnp.full((16,), b, jnp.int32)
            for k in range(4):
                plsc.store_scatter(t_v, [buf_c, eh_c[k], el_c[k], b_c], vs[k])

        def b_step(b8, vs):
            b = b8 * 8
            for db in range(8):
                vs_next = loads(b + db + 1)
                stores(b + db, vs)
                vs = vs_next
            return vs

        vs = loads(0)
        vs = lax.fori_loop(0, 15, b_step, vs)
        b = 120
        for db in range(7):
            vs_next = loads(b + db + 1)
            stores(b + db, vs)
            vs = vs_next
        stores(127, vs)

    def fire_gather(u, gbuf, sem):
        bb = u // L
        l = u - bb * L
        pltpu.async_copy(
            weight_hbm.at[xt_v.at[bb, l]],
            g_v.at[gbuf],
            sem,
        )

    def drain_gather(gbuf, sem):
        pltpu.make_async_copy(
            weight_hbm.at[pl.ds(0, 128)], g_v.at[gbuf], sem
        ).wait()

    def fire_scatter(u, tbuf, sem):
        bb = u // L
        l = u - bb * L
        pltpu.async_copy(
            t_v.at[tbuf, :, :, pl.ds(0, 128)],
            out_hbm.at[l, :, wid * BH_W + bb],
            sem,
        )

    def wait_scatter(u, tbuf, sem):
        bb = u // L
        l = u - bb * L
        pltpu.make_async_copy(
            t_v.at[tbuf, :, :, pl.ds(0, 128)],
            out_hbm.at[l, :, wid * BH_W + bb],
            sem,
        ).wait()

    fire_gather(0, 0, semg.at[0])
    fire_gather(1, 1, semg.at[1])

    drain_gather(0, semg.at[0])
    transpose_unit(0, 0)
    fire_gather(2, 0, semg.at[0])
    fire_scatter(0, 0, sems.at[0])
    drain_gather(1, semg.at[1])
    transpose_unit(1, 1)
    fire_gather(3, 1, semg.at[1])
    fire_scatter(1, 1, sems.at[1])

    def pair(t, _):
        u = 2 * t + 2
        drain_gather(0, semg.at[0])
        wait_scatter(u - 2, 0, sems.at[0])
        transpose_unit(0, 0)
        fire_gather(u + 2, 0, semg.at[0])
        fire_scatter(u, 0, sems.at[0])
        drain_gather(1, semg.at[1])
        wait_scatter(u - 1, 1, sems.at[1])
        transpose_unit(1, 1)
        fire_gather(u + 3, 1, semg.at[1])
        fire_scatter(u + 1, 1, sems.at[1])
        return 0

    lax.fori_loop(0, (NUNIT - 4) // 2, pair, 0)

    drain_gather(0, semg.at[0])
    wait_scatter(NUNIT - 4, 0, sems.at[0])
    transpose_unit(0, 0)
    fire_scatter(NUNIT - 2, 0, sems.at[0])
    drain_gather(1, semg.at[1])
    wait_scatter(NUNIT - 3, 1, sems.at[1])
    transpose_unit(1, 1)
    fire_scatter(NUNIT - 1, 1, sems.at[1])
    wait_scatter(NUNIT - 2, 0, sems.at[0])
    wait_scatter(NUNIT - 1, 1, sems.at[1])


@jax.jit
def kernel(x, weight):
    wp = jnp.pad(weight, ((0, 0), (0, EPAD - EMBED)))
    mesh = plsc.VectorSubcoreMesh(core_axis_name="c", subcore_axis_name="s")
    out = pl.kernel(
        _embed_body,
        out_type=jax.ShapeDtypeStruct((L, 8, BH, 8, 128), jnp.float32),
        mesh=mesh,
        scratch_types=[
            pltpu.VMEM((ROWS_W, L), jnp.int32),
            pltpu.VMEM((BH_W, L, 128), jnp.int32),
            pltpu.VMEM((2, 128, EPAD), jnp.float32),
            pltpu.VMEM((2, 8, 8, 129), jnp.float32),
            pltpu.SemaphoreType.DMA((2,)),
            pltpu.SemaphoreType.DMA((2,)),
        ],
        compiler_params=pltpu.CompilerParams(
            use_tc_tiling_on_sc=False, needs_layout_passes=False
        ),
    )(wp, x)
    return out.transpose(2, 4, 0, 1, 3).reshape(B, L, EMBED)


# speedup vs baseline: 1.7800x; 1.0393x over previous
import jax
import jax.numpy as jnp
from jax import lax
from jax.experimental import pallas as pl
from jax.experimental.pallas import tpu as pltpu
from jax.experimental.pallas import tpu_sc as plsc

VOCAB = 1000000
EMBED = 64
EPAD = 128
B = 16384
L = 50

NC = 2
NS = 16
NW = NC * NS

BH = B // 128
BH_W = BH // NW
ROWS_W = B // NW
NUNIT = BH_W * L


def _embed_body(weight_hbm, x_hbm, out_hbm, x_v, xt_v, g_v, t_v, semg, sems):
    wid = lax.axis_index("s") * NC + lax.axis_index("c")
    base = wid * ROWS_W
    lanes = lax.iota(jnp.int32, 16)

    pltpu.sync_copy(x_hbm.at[pl.ds(base, ROWS_W)], x_v)

    def idx_t(u, _):
        bb = u // L
        l = u - bb * L
        col = jnp.full((16,), l, jnp.int32)
        for b0 in range(8):
            rows = lanes + (bb * 128 + b0 * 16)
            v = plsc.load_gather(x_v, [rows, col])
            xt_v[bb, l, pl.ds(b0 * 16, 16)] = v + v
        return 0

    lax.fori_loop(0, NUNIT, idx_t, 0)

    def transpose_unit(gbuf, tbuf):
        buf_c = jnp.full((16,), tbuf, jnp.int32)
        eh_c = [(lanes + e0) // 8 for e0 in (0, 16, 32, 48)]
        el_c = [(lanes + e0) % 8 for e0 in (0, 16, 32, 48)]

        def loads(b):
            return [g_v[gbuf, b, pl.ds(k * 16, 16)] for k in range(4)]

        def stores(b, vs):
            b_c = jnp.full((16,), b, jnp.int32)
            for k in range(4):
                plsc.store_scatter(t_v, [buf_c, eh_c[k], el_c[k], b_c], vs[k])

        def b_step(b8, vs):
            b = b8 * 8
            for db in range(8):
                vs_next = loads(b + db + 1)
                stores(b + db, vs)
                vs = vs_next
            return vs

        vs = loads(0)
        vs = lax.fori_loop(0, 15, b_step, vs)
        b = 120
        for db in range(7):
            vs_next = loads(b + db + 1)
            stores(b + db, vs)
            vs = vs_next
        stores(127, vs)

    def fire_gather(u, gbuf, sem):
        bb = u // L
        l = u - bb * L
        pltpu.async_copy(
            weight_hbm.at[xt_v.at[bb, l]],
            g_v.at[gbuf],
            sem,
        )

    def drain_gather(gbuf, sem):
        pltpu.make_async_copy(
            weight_hbm.at[pl.ds(0, 128)], g_v.at[gbuf], sem
        ).wait()

    def fire_scatter(u, tbuf, sem):
        bb = u // L
        l = u - bb * L
        pltpu.async_copy(
            t_v.at[tbuf, :, :, pl.ds(0, 128)],
            out_hbm.at[l, :, wid * BH_W + bb],
            sem,
        )

    def wait_scatter(u, tbuf, sem):
        bb = u // L
        l = u - bb * L
        pltpu.make_async_copy(
            t_v.at[tbuf, :, :, pl.ds(0, 128)],
            out_hbm.at[l, :, wid * BH_W + bb],
            sem,
        ).wait()

    fire_gather(0, 0, semg.at[0])
    fire_gather(1, 1, semg.at[1])

    drain_gather(0, semg.at[0])
    transpose_unit(0, 0)
    fire_gather(2, 0, semg.at[0])
    fire_scatter(0, 0, sems.at[0])
    drain_gather(1, semg.at[1])
    transpose_unit(1, 1)
    fire_gather(3, 1, semg.at[1])
    fire_scatter(1, 1, sems.at[1])

    def pair(t, _):
        u = 2 * t + 2
        drain_gather(0, semg.at[0])
        wait_scatter(u - 2, 0, sems.at[0])
        transpose_unit(0, 0)
        fire_gather(u + 2, 0, semg.at[0])
        fire_scatter(u, 0, sems.at[0])
        drain_gather(1, semg.at[1])
        wait_scatter(u - 1, 1, sems.at[1])
        transpose_unit(1, 1)
        fire_gather(u + 3, 1, semg.at[1])
        fire_scatter(u + 1, 1, sems.at[1])
        return 0

    lax.fori_loop(0, (NUNIT - 4) // 2, pair, 0)

    drain_gather(0, semg.at[0])
    wait_scatter(NUNIT - 4, 0, sems.at[0])
    transpose_unit(0, 0)
    fire_scatter(NUNIT - 2, 0, sems.at[0])
    drain_gather(1, semg.at[1])
    wait_scatter(NUNIT - 3, 1, sems.at[1])
    transpose_unit(1, 1)
    fire_scatter(NUNIT - 1, 1, sems.at[1])
    wait_scatter(NUNIT - 2, 0, sems.at[0])
    wait_scatter(NUNIT - 1, 1, sems.at[1])


@jax.jit
def kernel(x, weight):
    wp = jnp.pad(weight, ((0, 0), (0, EPAD - EMBED)))
    mesh = plsc.VectorSubcoreMesh(core_axis_name="c", subcore_axis_name="s")
    out = pl.kernel(
        _embed_body,
        out_type=jax.ShapeDtypeStruct((L, 8, BH, 8, 128), jnp.float32),
        mesh=mesh,
        scratch_types=[
            pltpu.VMEM((ROWS_W, L), jnp.int32),
            pltpu.VMEM((BH_W, L, 128), jnp.int32),
            pltpu.VMEM((2, 128, EMBED), jnp.float32),
            pltpu.VMEM((2, 8, 8, 129), jnp.float32),
            pltpu.SemaphoreType.DMA((2,)),
            pltpu.SemaphoreType.DMA((2,)),
        ],
        compiler_params=pltpu.CompilerParams(
            use_tc_tiling_on_sc=False, needs_layout_passes=False
        ),
    )(wp.reshape(2 * VOCAB, EMBED), x)
    return out.transpose(2, 4, 0, 1, 3).reshape(B, L, EMBED)
